# baseline (speedup 1.0000x reference)
import functools

import jax
import jax.numpy as jnp
from jax import lax
from jax.experimental import pallas as pl
from jax.experimental.pallas import tpu as pltpu
from jax.experimental.pallas import tpu_sc as plsc

_N = 10000
_D = 128
_KP1 = 33
_KTOP = 32
_BR = 128
_NP = 10112
_GRID = _NP // _BR
_G = _NP // 128
_P = 6

_M = _N * _KP1
_NSUB = 16
_NWORK = 32
_ROWS_C = 162
_M_PAD = _NSUB * _ROWS_C * 128
_PER_W = _M_PAD // _NWORK
_VREGS_W = _PER_W // 16
_NIP = 10240

_NEG = -3e38


def _embed_body(x_ref, w1t_ref, b1_ref, w2t_ref, b2_ref, out_ref):
    h = jnp.dot(x_ref[...], w1t_ref[...], preferred_element_type=jnp.float32)
    h = jnp.maximum(h + b1_ref[...], 0.0)
    e = jnp.dot(h, w2t_ref[...], preferred_element_type=jnp.float32)
    e = e + b2_ref[...]
    nrm = jnp.sqrt(jnp.sum(e * e, axis=1, keepdims=True))
    out_ref[...] = e / jnp.maximum(nrm, 1e-12)


def _topk_body(en_ref, ent_ref, vals_ref, idx_ref,
               s3_ref, vs_ref, as_ref, m_ref, a_ref):
    i = pl.program_id(0)
    lane = lax.broadcasted_iota(jnp.int32, (_BR, 128), 1)
    cur = i % 2
    b = (i - 1) % 2

    def mm_slice(g0, g1, M, A):
        en = en_ref[...]
        for g in range(g0, min(g1, _G)):
            v = jnp.dot(en, ent_ref[:, g * 128:(g + 1) * 128],
                        preferred_element_type=jnp.float32)
            if g == _G - 1:
                v = jnp.where(lane >= _N - (_G - 1) * 128, _NEG, v)
            s3_ref[cur, g] = v
            gt = v > M
            M = jnp.where(gt, v, M)
            A = jnp.where(gt, g, A)
        return M, A

    def pass_scan(p, Mp):
        M = jnp.full((_BR, 128), _NEG, jnp.float32)
        A = jnp.zeros((_BR, 128), jnp.int32)
        for g in range(_G):
            v = s3_ref[b, g]
            cand = jnp.where(v < Mp, v, _NEG)
            gt = cand > M
            M = jnp.where(gt, cand, M)
            A = jnp.where(gt, g, A)
        vs_ref[p] = M
        as_ref[p] = A * 128 + lane
        return M

    nsl = -(-_G // (_P - 1))

    Mn = jnp.full((_BR, 128), _NEG, jnp.float32)
    An = jnp.zeros((_BR, 128), jnp.int32)
    Mp = m_ref[b]

    @pl.when(i > 0)
    def _():
        vs_ref[0] = Mp
        as_ref[0] = a_ref[b] * 128 + lane

    Mcur, Acur = Mn, An
    Mpp = Mp
    for p in range(1, _P):
        @pl.when(i < _GRID)
        def _(p=p, M=Mcur, A=Acur):
            M2, A2 = mm_slice((p - 1) * nsl, p * nsl, M, A)
            m_ref[cur] = M2
            a_ref[cur] = A2
        Mcur = m_ref[cur]
        Acur = a_ref[cur]

        @pl.when(i > 0)
        def _(p=p, Mpp=Mpp):
            pass_scan(p, Mpp)
        Mpp = vs_ref[p]

    def ebody(j, carry):
        ptr, vals, idxs = carry
        H = jnp.full((_BR, 128), _NEG, jnp.float32)
        GA = jnp.full((_BR, 128), 1 << 30, jnp.int32)
        for p in range(_P):
            selp = ptr == p
            H = jnp.where(selp, vs_ref[p], H)
            GA = jnp.where(selp, as_ref[p], GA)
        mrow = jnp.max(H, axis=1, keepdims=True)
        pickarr = jnp.where(H == mrow, GA, 1 << 30)
        amin = jnp.min(pickarr, axis=1, keepdims=True)
        adv = jnp.logical_and(H == mrow, GA == amin)
        ptr = ptr + adv.astype(jnp.int32)
        vals = jnp.where(lane == j, mrow, vals)
        idxs = jnp.where(lane == j, amin, idxs)
        return ptr, vals, idxs

    _, vals, idxs = lax.fori_loop(
        0, _KP1, ebody,
        (jnp.zeros((_BR, 128), jnp.int32),
         jnp.zeros((_BR, 128), jnp.float32),
         jnp.zeros((_BR, 128), jnp.int32)))
    nrow = jnp.sum(jnp.where(lane < _KP1, vals, 0.0), axis=1,
                   keepdims=True)
    vals_ref[...] = jnp.where(lane == _KP1, nrow, vals)
    idx_ref[...] = idxs


def _inv_body(nr_ref, nc0_ref, nc1_ref, out_ref):
    out_ref[...] = lax.rsqrt(nr_ref[...] + nc0_ref[...] + nc1_ref[...])


def _embed_call(x_pad, w1t, b1_2d, w2t, b2_2d):
    return pl.pallas_call(
        _embed_body,
        grid=(_GRID,),
        in_specs=[
            pl.BlockSpec((_BR, _D), lambda i: (i, 0)),
            pl.BlockSpec((_D, _D), lambda i: (0, 0)),
            pl.BlockSpec((1, _D), lambda i: (0, 0)),
            pl.BlockSpec((_D, _D), lambda i: (0, 0)),
            pl.BlockSpec((1, _D), lambda i: (0, 0)),
        ],
        out_specs=pl.BlockSpec((_BR, _D), lambda i: (i, 0)),
        out_shape=jax.ShapeDtypeStruct((_NP, _D), jnp.float32),
    )(x_pad, w1t, b1_2d, w2t, b2_2d)


def _topk_call(en, ent):
    return pl.pallas_call(
        _topk_body,
        grid=(_GRID + 1,),
        in_specs=[
            pl.BlockSpec((_BR, _D), lambda i: (jnp.minimum(i, _GRID - 1), 0)),
            pl.BlockSpec((_D, _NP), lambda i: (0, 0)),
        ],
        out_specs=[
            pl.BlockSpec((_BR, 128), lambda i: (jnp.maximum(i - 1, 0), 0)),
            pl.BlockSpec((_BR, 128), lambda i: (jnp.maximum(i - 1, 0), 0)),
        ],
        out_shape=[
            jax.ShapeDtypeStruct((_NP, 128), jnp.float32),
            jax.ShapeDtypeStruct((_NP, 128), jnp.int32),
        ],
        scratch_shapes=[
            pltpu.VMEM((2, _G, _BR, 128), jnp.float32),
            pltpu.VMEM((_P, _BR, 128), jnp.float32),
            pltpu.VMEM((_P, _BR, 128), jnp.int32),
            pltpu.VMEM((2, _BR, 128), jnp.float32),
            pltpu.VMEM((2, _BR, 128), jnp.int32),
        ],
    )(en, ent)


def _inv_call(nr2d, nc0_2d, nc1_2d):
    return pl.pallas_call(
        _inv_body,
        out_shape=jax.ShapeDtypeStruct((_NIP // 128, 128), jnp.float32),
    )(nr2d, nc0_2d, nc1_2d)


@functools.lru_cache(maxsize=None)
def _sc_kernels():
    mesh = plsc.VectorSubcoreMesh(core_axis_name="c", subcore_axis_name="s")

    @functools.partial(
        pl.kernel,
        mesh=mesh,
        out_type=jax.ShapeDtypeStruct((2, _N), jnp.float32),
        scratch_types=[
            pltpu.VMEM((_PER_W,), jnp.int32),
            pltpu.VMEM((_PER_W,), jnp.float32),
            pltpu.VMEM_SHARED((_N,), jnp.float32),
        ],
    )
    def _scatter_add_kernel(cols_hbm, vals_hbm, zeros_hbm, out_hbm,
                            cols_v, vals_v, shared):
        cid = lax.axis_index("c")
        sid = lax.axis_index("s")
        wid = sid * 2 + cid

        @pl.when(sid == 0)
        def _():
            pltpu.sync_copy(zeros_hbm, shared)

        plsc.subcore_barrier()
        pltpu.sync_copy(cols_hbm.at[wid], cols_v)
        pltpu.sync_copy(vals_hbm.at[wid], vals_v)
        pltpu.sync_copy(vals_v, shared.at[cols_v], add=True)
        plsc.subcore_barrier()

        @pl.when(sid == 0)
        def _():
            pltpu.sync_copy(shared, out_hbm.at[cid])

    @functools.partial(
        pl.kernel,
        mesh=mesh,
        out_type=jax.ShapeDtypeStruct((_NWORK, _PER_W), jnp.float32),
        scratch_types=[
            pltpu.VMEM((_PER_W,), jnp.int32),
            pltpu.VMEM((_PER_W,), jnp.int32),
            pltpu.VMEM((_PER_W,), jnp.float32),
            pltpu.VMEM((_PER_W,), jnp.float32),
            pltpu.VMEM((_PER_W,), jnp.float32),
            pltpu.VMEM_SHARED((_NIP,), jnp.float32),
            pltpu.SemaphoreType.DMA,
        ],
    )
    def _gather_norm_kernel(inv_hbm, rows_hbm, cols_hbm, vals_hbm, out_hbm,
                            rows_v, cols_v, vals_v, gr_v, gc_v, inv_s, sem):
        sid = lax.axis_index("s")
        wid = sid * 2 + lax.axis_index("c")

        @pl.when(sid == 0)
        def _():
            pltpu.sync_copy(inv_hbm, inv_s)

        pltpu.sync_copy(rows_hbm.at[wid], rows_v)
        pltpu.sync_copy(cols_hbm.at[wid], cols_v)
        pltpu.sync_copy(vals_hbm.at[wid], vals_v)
        plsc.subcore_barrier()
        cp_r = pltpu.async_copy(inv_s.at[rows_v], gr_v, sem)
        cp_c = pltpu.async_copy(inv_s.at[cols_v], gc_v, sem)
        cp_r.wait()
        cp_c.wait()

        def body(i, _):
            sl = pl.ds(i * 16, 16)
            gr_v[sl] = vals_v[sl] * gc_v[sl] * gr_v[sl]
            return 0

        lax.fori_loop(0, _VREGS_W, body, 0)
        pltpu.sync_copy(gr_v, out_hbm.at[wid])

    return _scatter_add_kernel, _gather_norm_kernel


def kernel(x, W1, b1, W2, b2, k):
    x_pad = jnp.pad(x, ((0, _NP - _N), (0, 0)))
    en = _embed_call(x_pad, W1.T, b1.reshape(1, _D), W2.T, b2.reshape(1, _D))
    ent = jnp.asarray(en.T)

    vals_all, idx_all = _topk_call(en, ent)
    vals33 = vals_all[:_N, :_KP1]
    cols33 = idx_all[:_N, :_KP1]
    nrow = vals_all[:_N, _KP1]

    rows_flat = jnp.repeat(
        jnp.arange(_N, dtype=jnp.int32), _KP1) + (k - _KTOP).astype(jnp.int32)
    cols_flat = cols33.reshape(-1)
    vals_flat = vals33.reshape(-1)

    pad_n = _M_PAD - _M
    pad_idx = (jnp.arange(pad_n, dtype=jnp.int32) % _N)
    cols_p = jnp.concatenate([cols_flat, pad_idx])
    rows_p = jnp.concatenate([rows_flat, pad_idx])
    vals_p = jnp.concatenate([vals_flat, jnp.zeros((pad_n,), jnp.float32)])

    _scatter_add_kernel, _gather_norm_kernel = _sc_kernels()
    norm_col2 = _scatter_add_kernel(
        cols_p.reshape(_NWORK, _PER_W),
        vals_p.reshape(_NWORK, _PER_W),
        jnp.zeros((_N,), jnp.float32),
    )

    nr_pad = jnp.pad(nrow, (0, _NIP - _N)).reshape(_NIP // 128, 128)
    nc0_pad = jnp.pad(norm_col2[0], (0, _NIP - _N)).reshape(_NIP // 128, 128)
    nc1_pad = jnp.pad(norm_col2[1], (0, _NIP - _N)).reshape(_NIP // 128, 128)
    inv = _inv_call(nr_pad, nc0_pad, nc1_pad).reshape(_NIP)

    out = _gather_norm_kernel(
        inv,
        rows_p.reshape(_NWORK, _PER_W),
        cols_p.reshape(_NWORK, _PER_W),
        vals_p.reshape(_NWORK, _PER_W),
    )
    values = out.reshape(-1)[:_M]

    edge_index = jnp.concatenate(
        [jnp.stack([rows_flat, cols_flat]), jnp.stack([cols_flat, rows_flat])],
        axis=0)
    edge_weight = jnp.concatenate([values, values])
    return edge_index, edge_weight


# speedup vs baseline: 1.0489x; 1.0489x over previous
import functools

import jax
import jax.numpy as jnp
from jax import lax
from jax.experimental import pallas as pl
from jax.experimental.pallas import tpu as pltpu
from jax.experimental.pallas import tpu_sc as plsc

_N = 10000
_D = 128
_KP1 = 33
_KTOP = 32
_BR = 128
_NP = 10112
_GRID = _NP // _BR
_G = _NP // 128
_P = 6

_M = _N * _KP1
_NSUB = 16
_NWORK = 32
_ROWS_C = 162
_M_PAD = _NSUB * _ROWS_C * 128
_PER_W = _M_PAD // _NWORK
_VREGS_W = _PER_W // 16
_NIP = 10240

_NEG = -3e38


def _embed_body(x_ref, w1t_ref, b1_ref, w2t_ref, b2_ref, out_ref):
    h = jnp.dot(x_ref[...], w1t_ref[...], preferred_element_type=jnp.float32)
    h = jnp.maximum(h + b1_ref[...], 0.0)
    e = jnp.dot(h, w2t_ref[...], preferred_element_type=jnp.float32)
    e = e + b2_ref[...]
    nrm = jnp.sqrt(jnp.sum(e * e, axis=1, keepdims=True))
    out_ref[...] = e / jnp.maximum(nrm, 1e-12)


def _topk_body(en_ref, ent_ref, vals_ref, idx_ref,
               s3_ref, vs_ref, as_ref, m_ref, a_ref):
    i = pl.program_id(0)
    lane = lax.broadcasted_iota(jnp.int32, (_BR, 128), 1)

    @pl.when(i < _GRID)
    def _():
        en = en_ref[...]
        M = jnp.full((_BR, 128), _NEG, jnp.float32)
        A = jnp.zeros((_BR, 128), jnp.int32)
        for g in range(_G):
            v = jnp.dot(en, ent_ref[:, g * 128:(g + 1) * 128],
                        preferred_element_type=jnp.float32)
            if g == _G - 1:
                v = jnp.where(lane >= _N - (_G - 1) * 128, _NEG, v)
            s3_ref[i % 2, g] = v
            gt = v > M
            M = jnp.where(gt, v, M)
            A = jnp.where(gt, g, A)
        m_ref[i % 2] = M
        a_ref[i % 2] = A

    @pl.when(i > 0)
    def _():
        b = (i - 1) % 2
        M = m_ref[b]
        vs_ref[0] = M
        as_ref[0] = a_ref[b] * 128 + lane

        def pass_body(p, Mp):
            M = jnp.full((_BR, 128), _NEG, jnp.float32)
            A = jnp.zeros((_BR, 128), jnp.int32)
            for g in range(_G):
                v = s3_ref[b, g]
                cand = jnp.where(v < Mp, v, _NEG)
                gt = cand > M
                M = jnp.where(gt, cand, M)
                A = jnp.where(gt, g, A)
            vs_ref[p] = M
            as_ref[p] = A * 128 + lane
            return M

        lax.fori_loop(1, _P, pass_body, M)

        def ebody(j, carry):
            ptr, vals, idxs = carry
            H = jnp.full((_BR, 128), _NEG, jnp.float32)
            GA = jnp.full((_BR, 128), 1 << 30, jnp.int32)
            for p in range(_P):
                selp = ptr == p
                H = jnp.where(selp, vs_ref[p], H)
                GA = jnp.where(selp, as_ref[p], GA)
            mrow = jnp.max(H, axis=1, keepdims=True)
            pickarr = jnp.where(H == mrow, GA, 1 << 30)
            amin = jnp.min(pickarr, axis=1, keepdims=True)
            adv = jnp.logical_and(H == mrow, GA == amin)
            ptr = ptr + adv.astype(jnp.int32)
            vals = jnp.where(lane == j, mrow, vals)
            idxs = jnp.where(lane == j, amin, idxs)
            return ptr, vals, idxs

        _, vals, idxs = lax.fori_loop(
            0, _KP1, ebody,
            (jnp.zeros((_BR, 128), jnp.int32),
             jnp.zeros((_BR, 128), jnp.float32),
             jnp.zeros((_BR, 128), jnp.int32)))
        nrow = jnp.sum(jnp.where(lane < _KP1, vals, 0.0), axis=1,
                       keepdims=True)
        vals_ref[...] = jnp.where(lane == _KP1, nrow, vals)
        idx_ref[...] = idxs


def _inv_body(nr_ref, nc0_ref, nc1_ref, out_ref):
    out_ref[...] = lax.rsqrt(nr_ref[...] + nc0_ref[...] + nc1_ref[...])


def _embed_call(x_pad, w1t, b1_2d, w2t, b2_2d):
    return pl.pallas_call(
        _embed_body,
        grid=(_GRID,),
        in_specs=[
            pl.BlockSpec((_BR, _D), lambda i: (i, 0)),
            pl.BlockSpec((_D, _D), lambda i: (0, 0)),
            pl.BlockSpec((1, _D), lambda i: (0, 0)),
            pl.BlockSpec((_D, _D), lambda i: (0, 0)),
            pl.BlockSpec((1, _D), lambda i: (0, 0)),
        ],
        out_specs=pl.BlockSpec((_BR, _D), lambda i: (i, 0)),
        out_shape=jax.ShapeDtypeStruct((_NP, _D), jnp.float32),
    )(x_pad, w1t, b1_2d, w2t, b2_2d)


def _topk_call(en, ent):
    return pl.pallas_call(
        _topk_body,
        grid=(_GRID + 1,),
        in_specs=[
            pl.BlockSpec((_BR, _D), lambda i: (jnp.minimum(i, _GRID - 1), 0)),
            pl.BlockSpec((_D, _NP), lambda i: (0, 0)),
        ],
        out_specs=[
            pl.BlockSpec((_BR, 128), lambda i: (jnp.maximum(i - 1, 0), 0)),
            pl.BlockSpec((_BR, 128), lambda i: (jnp.maximum(i - 1, 0), 0)),
        ],
        out_shape=[
            jax.ShapeDtypeStruct((_NP, 128), jnp.float32),
            jax.ShapeDtypeStruct((_NP, 128), jnp.int32),
        ],
        scratch_shapes=[
            pltpu.VMEM((2, _G, _BR, 128), jnp.float32),
            pltpu.VMEM((_P, _BR, 128), jnp.float32),
            pltpu.VMEM((_P, _BR, 128), jnp.int32),
            pltpu.VMEM((2, _BR, 128), jnp.float32),
            pltpu.VMEM((2, _BR, 128), jnp.int32),
        ],
    )(en, ent)


def _inv_call(nr2d, nc0_2d, nc1_2d):
    return pl.pallas_call(
        _inv_body,
        out_shape=jax.ShapeDtypeStruct((_NIP // 128, 128), jnp.float32),
    )(nr2d, nc0_2d, nc1_2d)


@functools.lru_cache(maxsize=None)
def _sc_kernels():
    mesh = plsc.VectorSubcoreMesh(core_axis_name="c", subcore_axis_name="s")

    @functools.partial(
        pl.kernel,
        mesh=mesh,
        out_type=jax.ShapeDtypeStruct((2, _N), jnp.float32),
        scratch_types=[
            pltpu.VMEM((_PER_W,), jnp.int32),
            pltpu.VMEM((_PER_W,), jnp.float32),
            pltpu.VMEM_SHARED((_N,), jnp.float32),
        ],
    )
    def _scatter_add_kernel(cols_hbm, vals_hbm, zeros_hbm, out_hbm,
                            cols_v, vals_v, shared):
        cid = lax.axis_index("c")
        sid = lax.axis_index("s")
        wid = sid * 2 + cid

        @pl.when(sid == 0)
        def _():
            pltpu.sync_copy(zeros_hbm, shared)

        plsc.subcore_barrier()
        pltpu.sync_copy(cols_hbm.at[wid], cols_v)
        pltpu.sync_copy(vals_hbm.at[wid], vals_v)
        pltpu.sync_copy(vals_v, shared.at[cols_v], add=True)
        plsc.subcore_barrier()

        @pl.when(sid == 0)
        def _():
            pltpu.sync_copy(shared, out_hbm.at[cid])

    @functools.partial(
        pl.kernel,
        mesh=mesh,
        out_type=jax.ShapeDtypeStruct((_NWORK, _PER_W), jnp.float32),
        scratch_types=[
            pltpu.VMEM((_PER_W,), jnp.int32),
            pltpu.VMEM((_PER_W,), jnp.int32),
            pltpu.VMEM((_PER_W,), jnp.float32),
            pltpu.VMEM((_PER_W,), jnp.float32),
            pltpu.VMEM((_PER_W,), jnp.float32),
            pltpu.VMEM_SHARED((_NIP,), jnp.float32),
            pltpu.SemaphoreType.DMA,
        ],
    )
    def _gather_norm_kernel(inv_hbm, rows_hbm, cols_hbm, vals_hbm, out_hbm,
                            rows_v, cols_v, vals_v, gr_v, gc_v, inv_s, sem):
        sid = lax.axis_index("s")
        wid = sid * 2 + lax.axis_index("c")

        @pl.when(sid == 0)
        def _():
            pltpu.sync_copy(inv_hbm, inv_s)

        pltpu.sync_copy(rows_hbm.at[wid], rows_v)
        pltpu.sync_copy(cols_hbm.at[wid], cols_v)
        pltpu.sync_copy(vals_hbm.at[wid], vals_v)
        plsc.subcore_barrier()
        cp_r = pltpu.async_copy(inv_s.at[rows_v], gr_v, sem)
        cp_c = pltpu.async_copy(inv_s.at[cols_v], gc_v, sem)
        cp_r.wait()
        cp_c.wait()

        def body(i, _):
            sl = pl.ds(i * 16, 16)
            gr_v[sl] = vals_v[sl] * gc_v[sl] * gr_v[sl]
            return 0

        lax.fori_loop(0, _VREGS_W, body, 0)
        pltpu.sync_copy(gr_v, out_hbm.at[wid])

    return _scatter_add_kernel, _gather_norm_kernel


def kernel(x, W1, b1, W2, b2, k):
    x_pad = jnp.pad(x, ((0, _NP - _N), (0, 0)))
    en = _embed_call(x_pad, W1.T, b1.reshape(1, _D), W2.T, b2.reshape(1, _D))
    ent = jnp.asarray(en.T)

    vals_all, idx_all = _topk_call(en, ent)
    vals33 = vals_all[:_N, :_KP1]
    cols33 = idx_all[:_N, :_KP1]
    nrow = vals_all[:_N, _KP1]

    rows_flat = jnp.repeat(
        jnp.arange(_N, dtype=jnp.int32), _KP1) + (k - _KTOP).astype(jnp.int32)
    cols_flat = cols33.reshape(-1)
    vals_flat = vals33.reshape(-1)

    pad_n = _M_PAD - _M
    pad_idx = (jnp.arange(pad_n, dtype=jnp.int32) % _N)
    cols_p = jnp.concatenate([cols_flat, pad_idx])
    rows_p = jnp.concatenate([rows_flat, pad_idx])
    vals_p = jnp.concatenate([vals_flat, jnp.zeros((pad_n,), jnp.float32)])

    _scatter_add_kernel, _gather_norm_kernel = _sc_kernels()
    norm_col2 = _scatter_add_kernel(
        cols_p.reshape(_NWORK, _PER_W),
        vals_p.reshape(_NWORK, _PER_W),
        jnp.zeros((_N,), jnp.float32),
    )

    nr_pad = jnp.pad(nrow, (0, _NIP - _N)).reshape(_NIP // 128, 128)
    nc0_pad = jnp.pad(norm_col2[0], (0, _NIP - _N)).reshape(_NIP // 128, 128)
    nc1_pad = jnp.pad(norm_col2[1], (0, _NIP - _N)).reshape(_NIP // 128, 128)
    inv = _inv_call(nr_pad, nc0_pad, nc1_pad).reshape(_NIP)

    out = _gather_norm_kernel(
        inv,
        rows_p.reshape(_NWORK, _PER_W),
        cols_p.reshape(_NWORK, _PER_W),
        vals_p.reshape(_NWORK, _PER_W),
    )
    values = out.reshape(-1)[:_M]

    edge_index = jnp.concatenate(
        [jnp.stack([rows_flat, cols_flat]), jnp.stack([cols_flat, rows_flat])],
        axis=0)
    edge_weight = jnp.concatenate([values, values])
    return edge_index, edge_weight
